# baseline (speedup 1.0000x reference)
import jax
import jax.numpy as jnp
from jax.experimental import pallas as pl

_N = 1000
_M = 256
_NNZ = 20000
_NCLASS = 64
_NUM_ADD = 2000
_TEMP = 0.5
_INT_MIN = -2147483648


def _select_body(s_ref, cnt_ref, sel_ref):
    s = s_ref[...] + 0.0
    incident = cnt_ref[...] > 0.0
    s = jnp.where(incident, jnp.float32(-2e30), s)
    bits = jax.lax.bitcast_convert_type(s, jnp.int32)
    mk = bits ^ (jax.lax.shift_right_arithmetic(bits, 31) & jnp.int32(0x7FFFFFFF))

    kf = jnp.float32(_NUM_ADD)

    def bisect_step(i, p):
        bitpos = jnp.int32(31) - i
        cand = p | jax.lax.shift_left(jnp.int32(1), bitpos)
        m_cand = cand ^ jnp.int32(_INT_MIN)
        cnt = jnp.sum((mk >= m_cand).astype(jnp.float32))
        return jnp.where(cnt >= kf, cand, p)

    p = jax.lax.fori_loop(0, 32, bisect_step, jnp.int32(0))
    tm = p ^ jnp.int32(_INT_MIN)

    gt = (mk > tm).astype(jnp.float32)
    tie = (mk == tm).astype(jnp.float32)
    need = kf - jnp.sum(gt)

    jj = jax.lax.broadcasted_iota(jnp.int32, (_M, _M), 0)
    kk = jax.lax.broadcasted_iota(jnp.int32, (_M, _M), 1)
    upper = (jj < kk).astype(jnp.float32)
    within = jnp.dot(tie, upper, preferred_element_type=jnp.float32)
    ii = jax.lax.broadcasted_iota(jnp.int32, (_N, _N), 0)
    ll = jax.lax.broadcasted_iota(jnp.int32, (_N, _N), 1)
    lower = (ll < ii).astype(jnp.float32)
    row_tot = jnp.sum(tie, axis=1, keepdims=True)
    base = jnp.dot(lower, row_tot, preferred_element_type=jnp.float32)
    rank = base + within
    sel = gt + tie * (rank < need).astype(jnp.float32)
    sel_ref[...] = sel


def _select_topk(S, cnt):
    return pl.pallas_call(
        _select_body,
        out_shape=jax.ShapeDtypeStruct((_N, _M), jnp.float32),
    )(S, cnt)


def kernel(X, H, V, E, W0, eps0, W1, eps1, Wout, eps_out, mlp1_w, mlp1_b, mlp2_w, mlp2_b, cos_weight):
    ones = jnp.ones((_NNZ,), jnp.float32)
    cnt = jax.ops.segment_sum(ones, V * _M + E, num_segments=_N * _M)
    cnt = cnt.reshape(_N, _M)
    E, cnt = jax.lax.optimization_barrier((E, cnt))

    c = jax.ops.segment_sum(ones, E, num_segments=_M)
    inv_c = jnp.maximum(c, 1.0)[:, None]

    def seg_mean(vals):
        s = jax.ops.segment_sum(vals, E, num_segments=_M)
        return s / inv_c

    def unigin(Xin, W, eps):
        Xe = seg_mean(Xin[V])
        Xv = jax.ops.segment_sum(Xe[E], V, num_segments=_N)
        return ((1.0 + eps) * Xin + Xv) @ W

    Xh = jax.nn.leaky_relu(unigin(X, W0, eps0))
    Xh = jax.nn.leaky_relu(unigin(Xh, W1, eps1))
    emb = jax.nn.leaky_relu(unigin(Xh, Wout, eps_out))

    eX = seg_mean(emb[V])

    def _l2norm(x):
        n = jnp.sqrt(jnp.sum(x * x, axis=-1, keepdims=True))
        return x / jnp.maximum(n, 1e-12)

    node_fc = jnp.transpose(_l2norm(emb[:, None, :] * cos_weight), (1, 0, 2))
    edge_fc = jnp.transpose(_l2norm(eX[:, None, :] * cos_weight), (1, 2, 0))
    S = jnp.matmul(node_fc, edge_fc).mean(axis=0)

    sel = _select_topk(S, cnt)

    slots = jnp.cumsum(sel.reshape(-1))
    ks = jnp.arange(1, _NUM_ADD + 1, dtype=jnp.float32)
    idx = jnp.searchsorted(slots, ks, side="left").astype(jnp.int32)
    row = idx // _M
    col = idx % _M

    combined = jnp.concatenate([emb[row], eX[col]], axis=-1)
    h1 = jax.nn.relu(combined @ mlp1_w + mlp1_b)
    prob = jax.nn.sigmoid((h1 @ mlp2_w + mlp2_b)[..., 0])

    u_full = jax.random.uniform(jax.random.key(42), (_N, _M), minval=1e-06, maxval=1.0 - 1e-06)
    u = u_full[row, col]
    logit = jnp.log(u) - jnp.log(1.0 - u) + jnp.log(prob + 1e-08) - jnp.log(1.0 - prob + 1e-08)
    soft = jax.nn.sigmoid(logit / _TEMP)
    hard = (soft > 0.5).astype(jnp.float32)

    vals = (H[row, col] + 1.0) * hard
    return jnp.zeros((_N, _M), jnp.float32).at[row, col].set(vals)


# speedup vs baseline: 1.0549x; 1.0549x over previous
import jax
import jax.numpy as jnp
from jax.experimental import pallas as pl

_N = 1000
_M = 256
_NNZ = 20000
_NCLASS = 64
_NUM_ADD = 2000
_TEMP = 0.5
_INT_MIN = -2147483648


def _select_body(s_ref, cnt_ref, sel_ref):
    s = s_ref[...] + 0.0
    incident = cnt_ref[...] > 0.0
    s = jnp.where(incident, jnp.float32(-2e30), s)
    bits = jax.lax.bitcast_convert_type(s, jnp.int32)
    mk = bits ^ (jax.lax.shift_right_arithmetic(bits, 31) & jnp.int32(0x7FFFFFFF))

    kf = jnp.float32(_NUM_ADD)

    def bisect_step(i, p):
        bitpos = jnp.int32(31) - i
        cand = p | jax.lax.shift_left(jnp.int32(1), bitpos)
        m_cand = cand ^ jnp.int32(_INT_MIN)
        cnt = jnp.sum((mk >= m_cand).astype(jnp.float32))
        return jnp.where(cnt >= kf, cand, p)

    p = jax.lax.fori_loop(0, 32, bisect_step, jnp.int32(0))
    tm = p ^ jnp.int32(_INT_MIN)

    gt = (mk > tm).astype(jnp.float32)
    tie = (mk == tm).astype(jnp.float32)
    need = kf - jnp.sum(gt)

    jj = jax.lax.broadcasted_iota(jnp.int32, (_M, _M), 0)
    kk = jax.lax.broadcasted_iota(jnp.int32, (_M, _M), 1)
    upper = (jj < kk).astype(jnp.float32)
    within = jnp.dot(tie, upper, preferred_element_type=jnp.float32)
    ii = jax.lax.broadcasted_iota(jnp.int32, (_N, _N), 0)
    ll = jax.lax.broadcasted_iota(jnp.int32, (_N, _N), 1)
    lower = (ll < ii).astype(jnp.float32)
    row_tot = jnp.sum(tie, axis=1, keepdims=True)
    base = jnp.dot(lower, row_tot, preferred_element_type=jnp.float32)
    rank = base + within
    sel = gt + tie * (rank < need).astype(jnp.float32)
    sel_ref[...] = sel


def _select_topk(S, cnt):
    return pl.pallas_call(
        _select_body,
        out_shape=jax.ShapeDtypeStruct((_N, _M), jnp.float32),
    )(S, cnt)


def kernel(X, H, V, E, W0, eps0, W1, eps1, Wout, eps_out, mlp1_w, mlp1_b, mlp2_w, mlp2_b, cos_weight):
    ones = jnp.ones((_NNZ,), jnp.float32)
    cnt = jax.ops.segment_sum(ones, V * _M + E, num_segments=_N * _M)
    cnt = cnt.reshape(_N, _M)
    E, cnt = jax.lax.optimization_barrier((E, cnt))

    c = jax.ops.segment_sum(ones, E, num_segments=_M)
    inv_c = jnp.maximum(c, 1.0)[:, None]

    def seg_mean(vals):
        s = jax.ops.segment_sum(vals, E, num_segments=_M)
        return s / inv_c

    def unigin(Xin, W, eps):
        Xe = seg_mean(Xin[V])
        Xv = jax.ops.segment_sum(Xe[E], V, num_segments=_N)
        return ((1.0 + eps) * Xin + Xv) @ W

    Xh = jax.nn.leaky_relu(unigin(X, W0, eps0))
    Xh = jax.nn.leaky_relu(unigin(Xh, W1, eps1))
    emb = jax.nn.leaky_relu(unigin(Xh, Wout, eps_out))

    eX = seg_mean(emb[V])

    def _l2norm(x):
        n = jnp.sqrt(jnp.sum(x * x, axis=-1, keepdims=True))
        return x / jnp.maximum(n, 1e-12)

    node_fc = jnp.transpose(_l2norm(emb[:, None, :] * cos_weight), (1, 0, 2))
    edge_fc = jnp.transpose(_l2norm(eX[:, None, :] * cos_weight), (1, 2, 0))
    S = jnp.matmul(node_fc, edge_fc).mean(axis=0)

    sel = _select_topk(S, cnt)

    combined = jnp.concatenate([
        jnp.broadcast_to(emb[:, None, :], (_N, _M, _NCLASS)),
        jnp.broadcast_to(eX[None, :, :], (_N, _M, _NCLASS)),
    ], axis=-1)
    h1 = jax.nn.relu(combined @ mlp1_w + mlp1_b)
    prob = jax.nn.sigmoid((h1 @ mlp2_w + mlp2_b)[..., 0])

    u = jax.random.uniform(jax.random.key(42), prob.shape, minval=1e-06, maxval=1.0 - 1e-06)
    logit = jnp.log(u) - jnp.log(1.0 - u) + jnp.log(prob + 1e-08) - jnp.log(1.0 - prob + 1e-08)
    soft = jax.nn.sigmoid(logit / _TEMP)
    hard = (soft > 0.5).astype(jnp.float32)

    return (H + sel) * hard
